# baseline (speedup 1.0000x reference)
import functools

import jax
import jax.numpy as jnp
from jax import lax
from jax.experimental import pallas as pl
from jax.experimental.pallas import tpu as pltpu
from jax.experimental.pallas import tpu_sc as plsc

EMB_DIM = 64
SCALE = 8.0
LANES = 16
CHUNK = 128
GROUPS = CHUNK // LANES


@functools.lru_cache(maxsize=None)
def _build(n_rows: int, n_pos: int):
    info = plsc.get_sparse_core_info()
    nc, ns = info.num_cores, info.num_subcores
    nw = nc * ns
    assert n_rows % nw == 0
    n_per_w = n_rows // nw
    assert n_per_w % (2 * CHUNK) == 0
    n_chunks = n_per_w // CHUNK
    n_pairs = n_chunks // 2

    mesh = plsc.VectorSubcoreMesh(core_axis_name="c", subcore_axis_name="s")

    @functools.partial(
        pl.kernel,
        out_type=jax.ShapeDtypeStruct((n_rows, EMB_DIM), jnp.float32),
        scratch_types=[
            pltpu.VMEM((n_per_w,), jnp.int32),
            pltpu.VMEM((n_per_w,), jnp.int32),
            pltpu.VMEM((n_pos, EMB_DIM), jnp.float32),
            pltpu.VMEM((CHUNK, EMB_DIM), jnp.float32),
            pltpu.VMEM((CHUNK, EMB_DIM), jnp.float32),
            pltpu.VMEM((CHUNK, EMB_DIM), jnp.float32),
            pltpu.VMEM((CHUNK, EMB_DIM), jnp.float32),
            pltpu.SemaphoreType.DMA,
            pltpu.SemaphoreType.DMA,
            pltpu.SemaphoreType.DMA,
            pltpu.SemaphoreType.DMA,
        ],
        mesh=mesh,
        compiler_params=pltpu.CompilerParams(use_tc_tiling_on_sc=True),
    )
    def fused(word_hbm, pos_hbm, table_hbm, emb1_hbm, out_hbm,
              widx, pidx, e1, ga, gb, oa, ob, sga, sgb, swa, swb):
        wid = lax.axis_index("s") * nc + lax.axis_index("c")
        w_base = wid * n_per_w

        pltpu.sync_copy(word_hbm.at[pl.ds(w_base, n_per_w)], widx)
        pltpu.sync_copy(pos_hbm.at[pl.ds(w_base, n_per_w)], pidx)
        pltpu.sync_copy(emb1_hbm, e1)

        def fire(ci, g, sem):
            cbase = ci * CHUNK

            def grp(gi, carry):
                wv = widx[pl.ds(cbase + gi * LANES, LANES)]
                for r in range(LANES):
                    i_s = wv[r]
                    pltpu.async_copy(
                        table_hbm.at[lax.shift_right_logical(i_s, 3), i_s & 7],
                        g.at[gi * LANES + r], sem)
                return carry

            lax.fori_loop(0, GROUPS, grp, 0, unroll=2)

        def wait_gather(g, sem):
            pltpu.make_async_copy(out_hbm.at[pl.ds(0, CHUNK)], g, sem).wait()

        def compute(ci, g, o):
            cbase = ci * CHUNK

            def group_body(gi, carry):
                rbase = gi * LANES
                pv = pidx[pl.ds(cbase + rbase, LANES)]
                for r in range(LANES):
                    p_s = pv[r]
                    for j in range(EMB_DIM // LANES):
                        csl = pl.ds(j * LANES, LANES)
                        o[rbase + r, csl] = g[rbase + r, csl] * SCALE + e1[p_s, csl]
                return carry

            lax.fori_loop(0, GROUPS, group_body, 0, unroll=2)

        def wb(ci, o, sem):
            base = w_base + ci * CHUNK
            return pltpu.async_copy(o, out_hbm.at[pl.ds(base, CHUNK)], sem)

        def wait_wb(o, sem):
            pltpu.make_async_copy(o, out_hbm.at[pl.ds(0, CHUNK)], sem).wait()

        fire(0, ga, sga)
        fire(1, gb, sgb)

        def pair_body(p, carry):
            ci = 2 * p
            wait_gather(ga, sga)
            compute(ci, ga, oa)
            fire(lax.min(ci + 2, n_chunks - 1), ga, sga)
            wb(ci, oa, swa)
            wait_gather(gb, sgb)
            compute(ci + 1, gb, ob)
            fire(lax.min(ci + 3, n_chunks - 1), gb, sgb)
            wb(ci + 1, ob, swb)
            wait_wb(oa, swa)
            wait_wb(ob, swb)
            return carry

        lax.fori_loop(0, n_pairs, pair_body, 0)

        wait_gather(ga, sga)
        wait_gather(gb, sgb)

    return fused


def kernel(src_word, src_pos, emb0_weight, emb1_weight):
    b, s = src_word.shape
    n_rows = b * s
    n_pos = emb1_weight.shape[0]
    n_vocab, d = emb0_weight.shape
    word = src_word.reshape(n_rows).astype(jnp.int32)
    pos = src_pos.reshape(n_rows).astype(jnp.int32)
    table3 = emb0_weight.reshape(n_vocab // 8, 8, d)
    out = _build(n_rows, n_pos)(word, pos, table3, emb1_weight)
    return out.reshape(b, s, EMB_DIM)


# speedup vs baseline: 1.0451x; 1.0451x over previous
import functools

import jax
import jax.numpy as jnp
from jax import lax
from jax.experimental import pallas as pl
from jax.experimental.pallas import tpu as pltpu
from jax.experimental.pallas import tpu_sc as plsc

EMB_DIM = 64
SCALE = 8.0
LANES = 16
CHUNK = 80
GROUPS = CHUNK // LANES
NBUF = 4


@functools.lru_cache(maxsize=None)
def _build(n_rows: int, n_pos: int):
    info = plsc.get_sparse_core_info()
    nc, ns = info.num_cores, info.num_subcores
    nw = nc * ns
    assert n_rows % nw == 0
    n_per_w = n_rows // nw
    assert n_per_w % (NBUF * CHUNK) == 0
    n_chunks = n_per_w // CHUNK
    n_quads = n_chunks // NBUF

    mesh = plsc.VectorSubcoreMesh(core_axis_name="c", subcore_axis_name="s")

    @functools.partial(
        pl.kernel,
        out_type=jax.ShapeDtypeStruct((n_rows, EMB_DIM), jnp.float32),
        scratch_types=(
            [pltpu.VMEM((n_per_w,), jnp.int32)] * 2
            + [pltpu.VMEM((n_pos, EMB_DIM), jnp.float32)]
            + [pltpu.VMEM((CHUNK, EMB_DIM), jnp.float32)] * NBUF
            + [pltpu.VMEM((CHUNK, EMB_DIM), jnp.float32)] * NBUF
            + [pltpu.SemaphoreType.DMA] * NBUF
            + [pltpu.SemaphoreType.DMA] * NBUF
        ),
        mesh=mesh,
        compiler_params=pltpu.CompilerParams(use_tc_tiling_on_sc=True),
    )
    def fused(word_hbm, pos_hbm, table_hbm, emb1_hbm, out_hbm,
              widx, pidx, e1, *bufs):
        g = bufs[0:NBUF]
        o = bufs[NBUF:2 * NBUF]
        sg = bufs[2 * NBUF:3 * NBUF]
        sw = bufs[3 * NBUF:4 * NBUF]
        wid = lax.axis_index("s") * nc + lax.axis_index("c")
        w_base = wid * n_per_w

        pltpu.sync_copy(word_hbm.at[pl.ds(w_base, n_per_w)], widx)
        pltpu.sync_copy(pos_hbm.at[pl.ds(w_base, n_per_w)], pidx)
        pltpu.sync_copy(emb1_hbm, e1)

        def fire_group(ci, gi, gbuf, sem):
            wv = widx[pl.ds(ci * CHUNK + gi * LANES, LANES)]
            hi = lax.shift_right_logical(wv, 3)
            lo = wv & 7
            for r in range(LANES):
                pltpu.async_copy(table_hbm.at[hi[r], lo[r]],
                                 gbuf.at[gi * LANES + r], sem)

        def fire(ci, gbuf, sem):
            def grp(gi, carry):
                fire_group(ci, gi, gbuf, sem)
                return carry
            lax.fori_loop(0, GROUPS, grp, 0)

        def wait_gather(gbuf, sem):
            pltpu.make_async_copy(out_hbm.at[pl.ds(0, CHUNK)], gbuf, sem).wait()

        def compute_and_fire(ci, gbuf, obuf, nci, ngbuf, nsem):
            cbase = ci * CHUNK

            def group_body(gi, carry):
                rbase = gi * LANES
                pv = pidx[pl.ds(cbase + rbase, LANES)]
                fire_group(nci, gi, ngbuf, nsem)
                for r in range(LANES):
                    p_s = pv[r]
                    for j in range(EMB_DIM // LANES):
                        csl = pl.ds(j * LANES, LANES)
                        obuf[rbase + r, csl] = (
                            gbuf[rbase + r, csl] * SCALE + e1[p_s, csl])
                return carry

            lax.fori_loop(0, GROUPS, group_body, 0)

        def wb(ci, obuf, sem):
            base = w_base + ci * CHUNK
            return pltpu.async_copy(obuf, out_hbm.at[pl.ds(base, CHUNK)], sem)

        def wait_wb(obuf, sem):
            pltpu.make_async_copy(obuf, out_hbm.at[pl.ds(0, CHUNK)], sem).wait()

        for b in range(NBUF - 1):
            fire(b, g[b], sg[b])

        def quad_body(q, carry):
            for b in range(NBUF):
                ci = q * NBUF + b
                nb = (b + NBUF - 1) % NBUF
                wait_gather(g[b], sg[b])

                @pl.when(q > 0)
                def _():
                    wait_wb(o[b], sw[b])

                compute_and_fire(ci, g[b], o[b],
                                 lax.min(ci + NBUF - 1, n_chunks - 1),
                                 g[nb], sg[nb])
                wb(ci, o[b], sw[b])
            return carry

        lax.fori_loop(0, n_quads, quad_body, 0)

        for b in range(NBUF - 1):
            wait_gather(g[b], sg[b])
        for b in range(NBUF):
            wait_wb(o[b], sw[b])

    return fused


def kernel(src_word, src_pos, emb0_weight, emb1_weight):
    b, s = src_word.shape
    n_rows = b * s
    n_pos = emb1_weight.shape[0]
    n_vocab, d = emb0_weight.shape
    word = src_word.reshape(n_rows).astype(jnp.int32)
    pos = src_pos.reshape(n_rows).astype(jnp.int32)
    table3 = emb0_weight.reshape(n_vocab // 8, 8, d)
    out = _build(n_rows, n_pos)(word, pos, table3, emb1_weight)
    return out.reshape(b, s, EMB_DIM)


# speedup vs baseline: 1.0649x; 1.0189x over previous
import functools

import jax
import jax.numpy as jnp
from jax import lax
from jax.experimental import pallas as pl
from jax.experimental.pallas import tpu as pltpu
from jax.experimental.pallas import tpu_sc as plsc

EMB_DIM = 64
SCALE = 8.0
LANES = 16
CHUNK = 80
GROUPS = CHUNK // LANES
NBUF = 4


@functools.lru_cache(maxsize=None)
def _build(n_rows: int, n_pos: int):
    info = plsc.get_sparse_core_info()
    nc, ns = info.num_cores, info.num_subcores
    nw = nc * ns
    assert n_rows % nw == 0
    n_per_w = n_rows // nw
    assert n_per_w % (NBUF * CHUNK) == 0
    n_chunks = n_per_w // CHUNK
    n_quads = n_chunks // NBUF

    mesh = plsc.VectorSubcoreMesh(core_axis_name="c", subcore_axis_name="s")

    @functools.partial(
        pl.kernel,
        out_type=jax.ShapeDtypeStruct((n_rows, EMB_DIM), jnp.float32),
        scratch_types=(
            [pltpu.VMEM((n_per_w,), jnp.int32)] * 2
            + [pltpu.VMEM((n_pos, EMB_DIM), jnp.float32)]
            + [pltpu.VMEM((CHUNK, EMB_DIM), jnp.float32)] * NBUF
            + [pltpu.VMEM((CHUNK, EMB_DIM), jnp.float32)] * NBUF
            + [pltpu.SemaphoreType.DMA] * NBUF
            + [pltpu.SemaphoreType.DMA] * NBUF
        ),
        mesh=mesh,
        compiler_params=pltpu.CompilerParams(use_tc_tiling_on_sc=True),
    )
    def fused(word_hbm, pos_hbm, table_hbm, emb1_hbm, out_hbm,
              widx, pidx, e1, *bufs):
        g = bufs[0:NBUF]
        o = bufs[NBUF:2 * NBUF]
        sg = bufs[2 * NBUF:3 * NBUF]
        sw = bufs[3 * NBUF:4 * NBUF]
        wid = lax.axis_index("s") * nc + lax.axis_index("c")
        w_base = wid * n_per_w

        pltpu.sync_copy(word_hbm.at[pl.ds(w_base, n_per_w)], widx)
        pltpu.sync_copy(pos_hbm.at[pl.ds(w_base, n_per_w)], pidx)
        pltpu.sync_copy(emb1_hbm, e1)

        def fire_group(ci, gi, gbuf, sem):
            wv = widx[pl.ds(ci * CHUNK + gi * LANES, LANES)]
            hi = lax.shift_right_logical(wv, 3)
            lo = wv & 7
            for r in range(LANES):
                pltpu.async_copy(table_hbm.at[hi[r], lo[r]],
                                 gbuf.at[gi * LANES + r], sem)

        def fire(ci, gbuf, sem):
            def grp(gi, carry):
                fire_group(ci, gi, gbuf, sem)
                return carry
            lax.fori_loop(0, GROUPS, grp, 0)

        def wait_gather(gbuf, sem):
            pltpu.make_async_copy(out_hbm.at[pl.ds(0, CHUNK)], gbuf, sem).wait()

        def compute_and_fire(ci, gbuf, obuf, nci, ngbuf, nsem):
            cbase = ci * CHUNK

            def group_body(gi, carry):
                rbase = gi * LANES
                pv = pidx[pl.ds(cbase + rbase, LANES)]
                wv = widx[pl.ds(nci * CHUNK + gi * LANES, LANES)]
                hi = lax.shift_right_logical(wv, 3)
                lo = wv & 7
                for r in range(LANES):
                    pltpu.async_copy(table_hbm.at[hi[r], lo[r]],
                                     ngbuf.at[gi * LANES + r], nsem)
                    p_s = pv[r]
                    for j in range(EMB_DIM // LANES):
                        csl = pl.ds(j * LANES, LANES)
                        obuf[rbase + r, csl] = (
                            gbuf[rbase + r, csl] * SCALE + e1[p_s, csl])
                return carry

            lax.fori_loop(0, GROUPS, group_body, 0)

        def wb(ci, obuf, sem):
            base = w_base + ci * CHUNK
            return pltpu.async_copy(obuf, out_hbm.at[pl.ds(base, CHUNK)], sem)

        def wait_wb(obuf, sem):
            pltpu.make_async_copy(obuf, out_hbm.at[pl.ds(0, CHUNK)], sem).wait()

        for b in range(NBUF - 1):
            fire(b, g[b], sg[b])

        def quad_body(q, carry):
            for b in range(NBUF):
                ci = q * NBUF + b
                nb = (b + NBUF - 1) % NBUF
                wait_gather(g[b], sg[b])

                @pl.when(q > 0)
                def _():
                    wait_wb(o[b], sw[b])

                compute_and_fire(ci, g[b], o[b],
                                 lax.min(ci + NBUF - 1, n_chunks - 1),
                                 g[nb], sg[nb])
                wb(ci, o[b], sw[b])
            return carry

        lax.fori_loop(0, n_quads, quad_body, 0)

        for b in range(NBUF - 1):
            wait_gather(g[b], sg[b])
        for b in range(NBUF):
            wait_wb(o[b], sw[b])

    return fused


def kernel(src_word, src_pos, emb0_weight, emb1_weight):
    b, s = src_word.shape
    n_rows = b * s
    n_pos = emb1_weight.shape[0]
    n_vocab, d = emb0_weight.shape
    word = src_word.reshape(n_rows).astype(jnp.int32)
    pos = src_pos.reshape(n_rows).astype(jnp.int32)
    table3 = emb0_weight.reshape(n_vocab // 8, 8, d)
    out = _build(n_rows, n_pos)(word, pos, table3, emb1_weight)
    return out.reshape(b, s, EMB_DIM)
